# baseline (speedup 1.0000x reference)
import jax
import jax.numpy as jnp
import numpy as np
from jax import lax
from jax.experimental import pallas as pl
from jax.experimental.pallas import tpu as pltpu
from jax.experimental.pallas import tpu_sc as plsc


def _build_signs():
    n = 12
    dim = 1 << n
    split_dims = tuple(reversed(range(n)))
    signs = np.empty(dim, dtype=np.float32)
    for split_index in range(dim):
        public_bits = [split_dims[b] for b in range(n) if split_index & (1 << b)]
        inv = 0
        for i, pi in enumerate(public_bits):
            for pj in public_bits[i + 1:]:
                if pi > pj:
                    inv += 1
        signs[split_index] = -1.0 if inv % 2 else 1.0
    return signs


_SIGNS_F32 = _build_signs()

D = 4096
ROWS = 4 * 2048
NC, NS = 2, 16
NW = NC * NS
ROWS_PER_W = ROWS // NW
RPG = 4
GSZ = RPG * D
GROUPS = ROWS_PER_W // RPG
VPG = RPG * 256


def _rev2(x):
    return ((x & 1) << 1) | ((x >> 1) & 1)


def _rev4(x):
    return ((x & 1) << 3) | ((x & 2) << 1) | ((x & 4) >> 1) | ((x & 8) >> 3)


def _body(mv_hbm, signs_hbm, out_hbm,
          signs_v, in0, in1, tbuf, o0, o1,
          sin0, sin1, sout0, sout1):
    wid = lax.axis_index("s") * NC + lax.axis_index("c")
    pltpu.sync_copy(signs_hbm, signs_v)
    base = wid * (ROWS_PER_W * D)
    inbuf = (in0, in1)
    outbuf = (o0, o1)
    sin = (sin0, sin1)
    sout = (sout0, sout1)

    lane = lax.iota(jnp.int32, 16)
    v1 = _rev2(lane >> 2) * 1024 + _rev2(lane & 3) * 4
    v2 = (lane >> 2) * 1024 + (lane & 3) * 4

    def start_in(g, b):
        pltpu.async_copy(mv_hbm.at[pl.ds(base + g * GSZ, GSZ)], inbuf[b], sin[b])

    def wait_in(b):
        pltpu.make_async_copy(mv_hbm.at[pl.ds(0, GSZ)], inbuf[b], sin[b]).wait()

    def start_out(g, b):
        pltpu.async_copy(outbuf[b], out_hbm.at[pl.ds(base + g * GSZ, GSZ)], sout[b])

    def wait_out(b):
        pltpu.make_async_copy(outbuf[b], out_hbm.at[pl.ds(0, GSZ)], sout[b]).wait()

    def compute(b):
        src, dst = inbuf[b], outbuf[b]

        @plsc.parallel_loop(0, VPG // 16, unroll=2)
        def _(j):
            r = j >> 4
            a0h = (j >> 2) & 3
            a1h = j & 3
            base1 = r * D + _rev2(a0h) * 256 + _rev2(a1h) * 16
            bv1 = base1 + v1
            for s in range(16):
                a1l, a2h = s >> 2, s & 3
                delta = _rev2(a1l) * 64 + _rev2(a2h)
                tbuf[pl.ds(j * 256 + s * 16, 16)] = plsc.load_gather(
                    src, [bv1 + delta]
                )

        @plsc.parallel_loop(0, VPG // 16, unroll=2)
        def _(j):
            r = j >> 4
            a2 = j & 15
            a2h, a2l = a2 >> 2, a2 & 3
            base2 = r * D + a2h * 16 + a2l
            bv2 = base2 + v2
            sbase = a2 * 256
            for s in range(16):
                sv = signs_v[pl.ds(sbase + s * 16, 16)]
                dst[pl.ds(j * 256 + s * 16, 16)] = (
                    plsc.load_gather(tbuf, [bv2 + s * 64]) * sv
                )

    start_in(0, 0)
    start_in(1, 1)
    for gg in (0, 1):
        b = gg & 1
        wait_in(b)
        compute(b)
        start_out(gg, b)
        start_in(gg + 2, b)

    @pl.loop(2, GROUPS - 2, step=2)
    def _(g):
        for bb in (0, 1):
            gg = g + bb
            wait_in(bb)
            wait_out(bb)
            compute(bb)
            start_out(gg, bb)
            start_in(gg + 2, bb)

    for gg in (GROUPS - 2, GROUPS - 1):
        b = gg & 1
        wait_in(b)
        wait_out(b)
        compute(b)
        start_out(gg, b)
    wait_out(0)
    wait_out(1)


@jax.jit
def _permute(mv_flat, signs):
    mesh = plsc.VectorSubcoreMesh(core_axis_name="c", subcore_axis_name="s")
    f = pl.kernel(
        _body,
        out_type=jax.ShapeDtypeStruct((ROWS * D,), jnp.float32),
        mesh=mesh,
        scratch_types=[
            pltpu.VMEM((D,), jnp.float32),
            pltpu.VMEM((GSZ,), jnp.float32),
            pltpu.VMEM((GSZ,), jnp.float32),
            pltpu.VMEM((GSZ,), jnp.float32),
            pltpu.VMEM((GSZ,), jnp.float32),
            pltpu.VMEM((GSZ,), jnp.float32),
            pltpu.SemaphoreType.DMA,
            pltpu.SemaphoreType.DMA,
            pltpu.SemaphoreType.DMA,
            pltpu.SemaphoreType.DMA,
        ],
        compiler_params=pltpu.CompilerParams(needs_layout_passes=False),
    )
    return f(mv_flat, signs)


def kernel(mv):
    mv_flat = mv.reshape(ROWS * D)
    out = _permute(mv_flat, jnp.asarray(_SIGNS_F32))
    return out.reshape(mv.shape)


# speedup vs baseline: 1.0437x; 1.0437x over previous
import jax
import jax.numpy as jnp
import numpy as np
from jax import lax
from jax.experimental import pallas as pl
from jax.experimental.pallas import tpu as pltpu
from jax.experimental.pallas import tpu_sc as plsc


def _build_signs():
    n = 12
    dim = 1 << n
    split_dims = tuple(reversed(range(n)))
    signs = np.empty(dim, dtype=np.float32)
    for split_index in range(dim):
        public_bits = [split_dims[b] for b in range(n) if split_index & (1 << b)]
        inv = 0
        for i, pi in enumerate(public_bits):
            for pj in public_bits[i + 1:]:
                if pi > pj:
                    inv += 1
        signs[split_index] = -1.0 if inv % 2 else 1.0
    return signs


_SIGNS_F32 = _build_signs()

D = 4096
ROWS = 4 * 2048
NC, NS = 2, 16
NW = NC * NS
ROWS_PER_W = ROWS // NW
RPG = 4
GSZ = RPG * D
GROUPS = ROWS_PER_W // RPG
VPG = RPG * 256


def _rev2(x):
    return ((x & 1) << 1) | ((x >> 1) & 1)


def _rev4(x):
    return ((x & 1) << 3) | ((x & 2) << 1) | ((x & 4) >> 1) | ((x & 8) >> 3)


def _body(mv_hbm, signs_hbm, out_hbm,
          signs_v, in0, in1, tbuf, o0, o1,
          sin0, sin1, sout0, sout1):
    wid = lax.axis_index("s") * NC + lax.axis_index("c")
    pltpu.sync_copy(signs_hbm, signs_v)
    base = wid * (ROWS_PER_W * D)
    inbuf = (in0, in1)
    outbuf = (o0, o1)
    sin = (sin0, sin1)
    sout = (sout0, sout1)

    lane = lax.iota(jnp.int32, 16)
    v1 = _rev2(lane & 3) * 16 + _rev2(lane >> 2)
    v2 = (lane >> 2) * 1024 + (lane & 3) * 4

    def start_in(g, b):
        pltpu.async_copy(mv_hbm.at[pl.ds(base + g * GSZ, GSZ)], inbuf[b], sin[b])

    def wait_in(b):
        pltpu.make_async_copy(mv_hbm.at[pl.ds(0, GSZ)], inbuf[b], sin[b]).wait()

    def start_out(g, b):
        pltpu.async_copy(outbuf[b], out_hbm.at[pl.ds(base + g * GSZ, GSZ)], sout[b])

    def wait_out(b):
        pltpu.make_async_copy(outbuf[b], out_hbm.at[pl.ds(0, GSZ)], sout[b]).wait()

    def compute(b):
        src, dst = inbuf[b], outbuf[b]

        @plsc.parallel_loop(0, VPG // 16)
        def _(j):
            r = j >> 4
            d2 = j & 15
            base1 = r * D + _rev2(d2 & 3) * 1024 + _rev2(d2 >> 2) * 4
            bv1 = base1 + v1
            for s in range(16):
                vals = src[pl.ds(j * 256 + s * 16, 16)]
                plsc.store_scatter(tbuf, [bv1 + _rev4(s) * 64], vals)

        @plsc.parallel_loop(0, VPG // 16)
        def _(j):
            r = j >> 4
            a2 = j & 15
            a2h, a2l = a2 >> 2, a2 & 3
            base2 = r * D + a2h * 16 + a2l
            bv2 = base2 + v2
            sbase = a2 * 256
            for s in range(16):
                sv = signs_v[pl.ds(sbase + s * 16, 16)]
                dst[pl.ds(j * 256 + s * 16, 16)] = (
                    plsc.load_gather(tbuf, [bv2 + s * 64]) * sv
                )

    start_in(0, 0)
    start_in(1, 1)
    for gg in (0, 1):
        b = gg & 1
        wait_in(b)
        compute(b)
        start_out(gg, b)
        start_in(gg + 2, b)

    @pl.loop(2, GROUPS - 2, step=2)
    def _(g):
        for bb in (0, 1):
            gg = g + bb
            wait_in(bb)
            wait_out(bb)
            compute(bb)
            start_out(gg, bb)
            start_in(gg + 2, bb)

    for gg in (GROUPS - 2, GROUPS - 1):
        b = gg & 1
        wait_in(b)
        wait_out(b)
        compute(b)
        start_out(gg, b)
    wait_out(0)
    wait_out(1)


@jax.jit
def _permute(mv_flat, signs):
    mesh = plsc.VectorSubcoreMesh(core_axis_name="c", subcore_axis_name="s")
    f = pl.kernel(
        _body,
        out_type=jax.ShapeDtypeStruct((ROWS * D,), jnp.float32),
        mesh=mesh,
        scratch_types=[
            pltpu.VMEM((D,), jnp.float32),
            pltpu.VMEM((GSZ,), jnp.float32),
            pltpu.VMEM((GSZ,), jnp.float32),
            pltpu.VMEM((GSZ,), jnp.float32),
            pltpu.VMEM((GSZ,), jnp.float32),
            pltpu.VMEM((GSZ,), jnp.float32),
            pltpu.SemaphoreType.DMA,
            pltpu.SemaphoreType.DMA,
            pltpu.SemaphoreType.DMA,
            pltpu.SemaphoreType.DMA,
        ],
        compiler_params=pltpu.CompilerParams(needs_layout_passes=False),
    )
    return f(mv_flat, signs)


def kernel(mv):
    mv_flat = mv.reshape(ROWS * D)
    out = _permute(mv_flat, jnp.asarray(_SIGNS_F32))
    return out.reshape(mv.shape)
